# baseline (speedup 1.0000x reference)
import jax
import jax.numpy as jnp
from jax import lax
from jax.experimental import pallas as pl
from jax.experimental.pallas import tpu as pltpu
from jax.experimental.pallas import tpu_sc as plsc

E = 8
K = 2
D = 768
BS = 256
DFFN = 1536
T = 2048
NPAIR = T * K
NB = 23
P = NB * BS
TBLK = 128
NTB = T // TBLK
NW = 32


def _router_body(x_ref, rwin_ref, sel0_ref, sel1_ref, rank0_ref, rank1_ref,
                 rw0_ref, rw1_ref, c0_ref, offs_ref, nblk_ref,
                 carry0, carry1):
    i = pl.program_id(0)

    @pl.when(i == 0)
    def _():
        carry0[...] = jnp.zeros_like(carry0)
        carry1[...] = jnp.zeros_like(carry1)

    xb = x_ref[...]
    logits = lax.dot_general(rwin_ref[...], xb, (((1,), (1,)), ((), ())),
                             preferred_element_type=jnp.float32)
    m = jnp.max(logits, axis=0, keepdims=True)
    ex = jnp.exp(logits - m)
    p = ex / jnp.sum(ex, axis=0, keepdims=True)
    sub8 = lax.broadcasted_iota(jnp.int32, (E, TBLK), 0)
    m1 = jnp.max(p, axis=0, keepdims=True)
    i1 = jnp.min(jnp.where(p >= m1, sub8, E), axis=0, keepdims=True)
    p2 = jnp.where(sub8 == i1, -1.0, p)
    m2 = jnp.max(p2, axis=0, keepdims=True)
    i2 = jnp.min(jnp.where(p2 >= m2, sub8, E), axis=0, keepdims=True)
    ssum = m1 + m2
    sel0_ref[...] = i1.reshape(TBLK)
    sel1_ref[...] = i2.reshape(TBLK)
    rw0_ref[...] = (m1 / ssum).reshape(TBLK)
    rw1_ref[...] = (m2 / ssum).reshape(TBLK)

    row128 = lax.broadcasted_iota(jnp.int32, (128, 128), 0)
    col128 = lax.broadcasted_iota(jnp.int32, (128, 128), 1)
    triu_incl = (row128 <= col128).astype(jnp.float32)
    for sel, carry, rref in ((i1, carry0, rank0_ref), (i2, carry1, rank1_ref)):
        oh = (row128 == sel).astype(jnp.float32)
        cum = jnp.dot(oh, triu_incl, preferred_element_type=jnp.float32)
        cb = carry[...]
        rank = jnp.sum(oh * (cum + cb - 1.0), axis=0, keepdims=True)
        rref[...] = rank.astype(jnp.int32).reshape(TBLK)
        carry[...] = cb + jnp.sum(oh, axis=1, keepdims=True)

    @pl.when(i == NTB - 1)
    def _():
        c0c = carry0[...]
        tot = (c0c + carry1[...]).astype(jnp.int32)
        padded = ((tot + BS - 1) >> 8) << 8
        nblk = (padded >> 8).astype(jnp.float32)
        low_strict = (row128 > col128).astype(jnp.float32)
        offs_col = jnp.dot(low_strict, padded.astype(jnp.float32),
                           preferred_element_type=jnp.float32)
        eye = (row128 == col128).astype(jnp.float32)
        c0_ref[...] = jnp.sum(eye * c0c, axis=0).astype(jnp.int32)
        offs_ref[...] = jnp.sum(eye * offs_col, axis=0).astype(jnp.int32)
        nblk_ref[...] = jnp.sum(eye * nblk, axis=0).astype(jnp.int32)


def _router(x2d, router_w):
    flat_i = jax.ShapeDtypeStruct((T,), jnp.int32)
    flat_f = jax.ShapeDtypeStruct((T,), jnp.float32)
    meta_i = jax.ShapeDtypeStruct((128,), jnp.int32)
    blk = pl.BlockSpec((TBLK,), lambda i: (i,))
    meta = pl.BlockSpec((128,), lambda i: (0,))
    return pl.pallas_call(
        _router_body,
        grid=(NTB,),
        in_specs=[
            pl.BlockSpec((TBLK, D), lambda i: (i, 0)),
            pl.BlockSpec((E, D), lambda i: (0, 0)),
        ],
        out_specs=[blk, blk, blk, blk, blk, blk, meta, meta, meta],
        out_shape=[flat_i, flat_i, flat_i, flat_i, flat_f, flat_f,
                   meta_i, meta_i, meta_i],
        scratch_shapes=[
            pltpu.VMEM((128, 1), jnp.float32),
            pltpu.VMEM((128, 1), jnp.float32),
        ],
    )(x2d, router_w)


def _dispatch_body(sel0_h, sel1_h, rank0_h, rank1_h, c0_h, offs_h, x_h,
                   xs_h, dst_h, selc_v, rankc_v, c0_v, offs_v,
                   dst0_v, dst1_v, tok0_v, tok1_v, rows0_v, rows1_v,
                   semg0, semg1, sems0, sems1):
    wid = lax.axis_index("s") * 2 + lax.axis_index("c")
    kflag = wid // 16
    tb = (wid % 16) * 128
    pb = wid * 128

    @pl.when(kflag == 0)
    def _():
        pltpu.sync_copy(sel0_h.at[pl.ds(tb, 128)], selc_v)
        pltpu.sync_copy(rank0_h.at[pl.ds(tb, 128)], rankc_v)

    @pl.when(kflag == 1)
    def _():
        pltpu.sync_copy(sel1_h.at[pl.ds(tb, 128)], selc_v)
        pltpu.sync_copy(rank1_h.at[pl.ds(tb, 128)], rankc_v)

    pltpu.sync_copy(c0_h.at[pl.ds(0, 16)], c0_v)
    pltpu.sync_copy(offs_h.at[pl.ds(0, 16)], offs_v)
    kvec = jnp.full((16,), kflag, dtype=jnp.int32)
    for j in range(8):
        dvh, tvh = (dst0_v, tok0_v) if j < 4 else (dst1_v, tok1_v)
        jh = j % 4
        s16 = selc_v[pl.ds(j * 16, 16)]
        r16 = rankc_v[pl.ds(j * 16, 16)]
        o16 = plsc.load_gather(offs_v, [s16])
        c016 = plsc.load_gather(c0_v, [s16])
        dvh[pl.ds(jh * 16, 16)] = o16 + c016 * kvec + r16
        tvh[pl.ds(jh * 16, 16)] = tb + j * 16 + lax.iota(jnp.int32, 16)
    g0 = pltpu.async_copy(x_h.at[tok0_v], rows0_v, semg0)
    g1 = pltpu.async_copy(x_h.at[tok1_v], rows1_v, semg1)
    g0.wait()
    s0 = pltpu.async_copy(rows0_v, xs_h.at[dst0_v], sems0)
    g1.wait()
    s1 = pltpu.async_copy(rows1_v, xs_h.at[dst1_v], sems1)
    pltpu.sync_copy(dst0_v, dst_h.at[pl.ds(pb, 64)])
    pltpu.sync_copy(dst1_v, dst_h.at[pl.ds(pb + 64, 64)])
    s0.wait()
    s1.wait()


def _dispatch(sel0, sel1, rank0, rank1, c0v, offsv, x2d):
    f = pl.kernel(
        _dispatch_body,
        out_type=[
            jax.ShapeDtypeStruct((P, D), jnp.float32),
            jax.ShapeDtypeStruct((NPAIR,), jnp.int32),
        ],
        mesh=plsc.VectorSubcoreMesh(core_axis_name="c", subcore_axis_name="s"),
        compiler_params=pltpu.CompilerParams(needs_layout_passes=False),
        scratch_types=[
            pltpu.VMEM((128,), jnp.int32),
            pltpu.VMEM((128,), jnp.int32),
            pltpu.VMEM((16,), jnp.int32),
            pltpu.VMEM((16,), jnp.int32),
            pltpu.VMEM((64,), jnp.int32),
            pltpu.VMEM((64,), jnp.int32),
            pltpu.VMEM((64,), jnp.int32),
            pltpu.VMEM((64,), jnp.int32),
            pltpu.VMEM((64, D), jnp.float32),
            pltpu.VMEM((64, D), jnp.float32),
            pltpu.SemaphoreType.DMA,
            pltpu.SemaphoreType.DMA,
            pltpu.SemaphoreType.DMA,
            pltpu.SemaphoreType.DMA,
        ],
    )
    return f(sel0, sel1, rank0, rank1, c0v, offsv, x2d)


def _ffn_body(nb_ref, xs_hbm, w1_hbm, w2_hbm, y_hbm, w1b, w2b, xsb, yb,
              w1sa, w1sb, w2sa, w2sb, xss, yss):
    H = DFFN // 2

    class _Pair:
        def __init__(self, a, b):
            self.a, self.b = a, b

        def start(self):
            self.a.start()
            self.b.start()

        def wait(self):
            self.a.wait()
            self.b.wait()

    def w1cp(e, slot):
        return _Pair(
            pltpu.make_async_copy(w1_hbm.at[:, pl.ds(e * DFFN, H)],
                                  w1b.at[slot, :, pl.ds(0, H)],
                                  w1sa.at[slot]),
            pltpu.make_async_copy(w1_hbm.at[:, pl.ds(e * DFFN + H, H)],
                                  w1b.at[slot, :, pl.ds(H, H)],
                                  w1sb.at[slot]))

    def w2cp(e, slot):
        return _Pair(
            pltpu.make_async_copy(w2_hbm.at[pl.ds(e * DFFN, H), :],
                                  w2b.at[slot, pl.ds(0, H)],
                                  w2sa.at[slot]),
            pltpu.make_async_copy(w2_hbm.at[pl.ds(e * DFFN + H, H), :],
                                  w2b.at[slot, pl.ds(H, H)],
                                  w2sb.at[slot]))

    def xscp(g, slot):
        return pltpu.make_async_copy(
            xs_hbm.at[pl.ds(g * BS, BS)], xsb.at[slot], xss.at[slot])

    def ycp(g, slot):
        return pltpu.make_async_copy(
            yb.at[slot], y_hbm.at[pl.ds(g * BS, BS)], yss.at[slot])

    nbtot = nb_ref[0]
    for e in range(1, E):
        nbtot = nbtot + nb_ref[e]

    for e in range(4):
        w1cp(e, e).start()
        w2cp(e, e).start()
    xscp(0, 0).start()

    g = 0
    for e in range(E):
        slot = e % 4
        w1cp(e, slot).wait()
        w2cp(e, slot).wait()

        def blk(j, g, slot=slot):
            xslot = lax.rem(g, 2)
            xscp(g, xslot).wait()

            @pl.when(g + 1 < nbtot)
            def _():
                xscp(g + 1, lax.rem(g + 1, 2)).start()

            @pl.when(g >= 2)
            def _():
                ycp(g - 2, xslot).wait()

            h = jnp.dot(xsb[xslot].astype(jnp.bfloat16),
                        w1b[slot].astype(jnp.bfloat16),
                        preferred_element_type=jnp.float32)
            h = jax.nn.gelu(h)
            yb[xslot] = jnp.dot(h.astype(jnp.bfloat16),
                                w2b[slot].astype(jnp.bfloat16),
                                preferred_element_type=jnp.float32)
            ycp(g, xslot).start()
            return g + 1

        g = lax.fori_loop(0, nb_ref[e], blk, g)
        if e + 4 < E:
            w1cp(e + 4, slot).start()
            w2cp(e + 4, slot).start()

    ycp(g - 1, lax.rem(g - 1, 2)).wait()
    ycp(g - 2, lax.rem(g - 2, 2)).wait()


def _ffn(nblk, xs, w1, w2):
    return pl.pallas_call(
        _ffn_body,
        in_specs=[
            pl.BlockSpec(memory_space=pltpu.SMEM),
            pl.BlockSpec(memory_space=pl.ANY),
            pl.BlockSpec(memory_space=pl.ANY),
            pl.BlockSpec(memory_space=pl.ANY),
        ],
        out_specs=pl.BlockSpec(memory_space=pl.ANY),
        out_shape=jax.ShapeDtypeStruct((P, D), jnp.float32),
        scratch_shapes=[
            pltpu.VMEM((4, D, DFFN), jnp.float32),
            pltpu.VMEM((4, DFFN, D), jnp.float32),
            pltpu.VMEM((2, BS, D), jnp.float32),
            pltpu.VMEM((2, BS, D), jnp.float32),
            pltpu.SemaphoreType.DMA((4,)),
            pltpu.SemaphoreType.DMA((4,)),
            pltpu.SemaphoreType.DMA((4,)),
            pltpu.SemaphoreType.DMA((4,)),
            pltpu.SemaphoreType.DMA((2,)),
            pltpu.SemaphoreType.DMA((2,)),
        ],
    )(nblk, xs, w1, w2)


def _combine_body(y_h, dst_h, rw0_h, rw1_h, o_h, i0a_v, i1a_v, i0b_v, i1b_v,
                  w0_v, w1_v, r0a_v, r1a_v, r0b_v, r1b_v,
                  sga0, sga1, sgb0, sgb1, soa, sob):
    wid = lax.axis_index("s") * 2 + lax.axis_index("c")
    tb = wid * 64
    pltpu.sync_copy(dst_h.at[pl.ds(tb, 32)], i0a_v)
    pltpu.sync_copy(dst_h.at[pl.ds(T + tb, 32)], i1a_v)
    pltpu.sync_copy(dst_h.at[pl.ds(tb + 32, 32)], i0b_v)
    pltpu.sync_copy(dst_h.at[pl.ds(T + tb + 32, 32)], i1b_v)
    pltpu.sync_copy(rw0_h.at[pl.ds(tb, 64)], w0_v)
    pltpu.sync_copy(rw1_h.at[pl.ds(tb, 64)], w1_v)
    ga0 = pltpu.async_copy(y_h.at[i0a_v], r0a_v, sga0)
    ga1 = pltpu.async_copy(y_h.at[i1a_v], r1a_v, sga1)
    gb0 = pltpu.async_copy(y_h.at[i0b_v], r0b_v, sgb0)
    gb1 = pltpu.async_copy(y_h.at[i1b_v], r1b_v, sgb1)

    def combine(r0_v, r1_v, woff):
        def body(j, carry):
            jw = jnp.full((16,), j + woff, dtype=jnp.int32)
            w0 = plsc.load_gather(w0_v, [jw])
            w1s = plsc.load_gather(w1_v, [jw])
            for c in range(D // 16):
                sl = pl.ds(c * 16, 16)
                r0_v[j, sl] = r0_v[j, sl] * w0 + r1_v[j, sl] * w1s
            return carry
        lax.fori_loop(0, 32, body, 0)

    ga0.wait()
    ga1.wait()
    combine(r0a_v, r1a_v, 0)
    oa = pltpu.async_copy(r0a_v, o_h.at[pl.ds(tb, 32)], soa)
    gb0.wait()
    gb1.wait()
    combine(r0b_v, r1b_v, 32)
    ob = pltpu.async_copy(r0b_v, o_h.at[pl.ds(tb + 32, 32)], sob)
    oa.wait()
    ob.wait()


def _combine(y, dst, rw0f, rw1f):
    f = pl.kernel(
        _combine_body,
        out_type=jax.ShapeDtypeStruct((T, D), jnp.float32),
        mesh=plsc.VectorSubcoreMesh(core_axis_name="c", subcore_axis_name="s"),
        compiler_params=pltpu.CompilerParams(needs_layout_passes=False),
        scratch_types=[
            pltpu.VMEM((32,), jnp.int32),
            pltpu.VMEM((32,), jnp.int32),
            pltpu.VMEM((32,), jnp.int32),
            pltpu.VMEM((32,), jnp.int32),
            pltpu.VMEM((64,), jnp.float32),
            pltpu.VMEM((64,), jnp.float32),
            pltpu.VMEM((32, D), jnp.float32),
            pltpu.VMEM((32, D), jnp.float32),
            pltpu.VMEM((32, D), jnp.float32),
            pltpu.VMEM((32, D), jnp.float32),
            pltpu.SemaphoreType.DMA,
            pltpu.SemaphoreType.DMA,
            pltpu.SemaphoreType.DMA,
            pltpu.SemaphoreType.DMA,
            pltpu.SemaphoreType.DMA,
            pltpu.SemaphoreType.DMA,
        ],
    )
    return f(y, dst, rw0f, rw1f)


def kernel(x, router_w, w1, w2):
    b, s, d = x.shape
    x2d = x.reshape(T, D)
    (sel0, sel1, rank0, rank1, rw0f, rw1f,
     c0v, offsv, nblkv) = _router(x2d, router_w)
    xs, dst = _dispatch(sel0, sel1, rank0, rank1, c0v, offsv, x2d)
    y = _ffn(nblkv, xs, w1, w2)
    out = _combine(y, dst, rw0f, rw1f)
    return out.reshape(b, s, d)


# speedup vs baseline: 1.0387x; 1.0387x over previous
import jax
import jax.numpy as jnp
from jax import lax
from jax.experimental import pallas as pl
from jax.experimental.pallas import tpu as pltpu
from jax.experimental.pallas import tpu_sc as plsc

E = 8
K = 2
D = 768
BS = 256
DFFN = 1536
T = 2048
NPAIR = T * K
NB = 23
P = NB * BS
TBLK = 256
NTB = T // TBLK
NW = 32


def _router_body(x_ref, rwin_ref, sel0_ref, sel1_ref, rank0_ref, rank1_ref,
                 rw0_ref, rw1_ref, c0_ref, offs_ref, nblk_ref,
                 carry0, carry1):
    i = pl.program_id(0)

    @pl.when(i == 0)
    def _():
        carry0[...] = jnp.zeros_like(carry0)
        carry1[...] = jnp.zeros_like(carry1)

    xb = x_ref[...]
    logits = lax.dot_general(rwin_ref[...], xb, (((1,), (1,)), ((), ())),
                             preferred_element_type=jnp.float32)
    m = jnp.max(logits, axis=0, keepdims=True)
    ex = jnp.exp(logits - m)
    p = ex / jnp.sum(ex, axis=0, keepdims=True)
    sub8 = lax.broadcasted_iota(jnp.int32, (E, TBLK), 0)
    m1 = jnp.max(p, axis=0, keepdims=True)
    i1 = jnp.min(jnp.where(p >= m1, sub8, E), axis=0, keepdims=True)
    p2 = jnp.where(sub8 == i1, -1.0, p)
    m2 = jnp.max(p2, axis=0, keepdims=True)
    i2 = jnp.min(jnp.where(p2 >= m2, sub8, E), axis=0, keepdims=True)
    ssum = m1 + m2
    sel0_ref[...] = i1.reshape(TBLK)
    sel1_ref[...] = i2.reshape(TBLK)
    rw0_ref[...] = (m1 / ssum).reshape(TBLK)
    rw1_ref[...] = (m2 / ssum).reshape(TBLK)

    rowE = lax.broadcasted_iota(jnp.int32, (128, TBLK), 0)
    rowT = lax.broadcasted_iota(jnp.int32, (TBLK, TBLK), 0)
    colT = lax.broadcasted_iota(jnp.int32, (TBLK, TBLK), 1)
    triu_incl = (rowT <= colT).astype(jnp.float32)
    for sel, carry, rref in ((i1, carry0, rank0_ref), (i2, carry1, rank1_ref)):
        oh = (rowE == sel).astype(jnp.float32)
        cum = jnp.dot(oh, triu_incl, preferred_element_type=jnp.float32)
        cb = carry[...]
        rank = jnp.sum(oh * (cum + cb - 1.0), axis=0, keepdims=True)
        rref[...] = rank.astype(jnp.int32).reshape(TBLK)
        carry[...] = cb + jnp.sum(oh, axis=1, keepdims=True)

    @pl.when(i == NTB - 1)
    def _():
        row128 = lax.broadcasted_iota(jnp.int32, (128, 128), 0)
        col128 = lax.broadcasted_iota(jnp.int32, (128, 128), 1)
        c0c = carry0[...]
        tot = (c0c + carry1[...]).astype(jnp.int32)
        padded = ((tot + BS - 1) >> 8) << 8
        nblk = (padded >> 8).astype(jnp.float32)
        low_strict = (row128 > col128).astype(jnp.float32)
        offs_col = jnp.dot(low_strict, padded.astype(jnp.float32),
                           preferred_element_type=jnp.float32)
        eye = (row128 == col128).astype(jnp.float32)
        c0_ref[...] = jnp.sum(eye * c0c, axis=0).astype(jnp.int32)
        offs_ref[...] = jnp.sum(eye * offs_col, axis=0).astype(jnp.int32)
        nblk_ref[...] = jnp.sum(eye * nblk, axis=0).astype(jnp.int32)


def _router(x2d, router_w):
    flat_i = jax.ShapeDtypeStruct((T,), jnp.int32)
    flat_f = jax.ShapeDtypeStruct((T,), jnp.float32)
    meta_i = jax.ShapeDtypeStruct((128,), jnp.int32)
    blk = pl.BlockSpec((TBLK,), lambda i: (i,))
    meta = pl.BlockSpec((128,), lambda i: (0,))
    return pl.pallas_call(
        _router_body,
        grid=(NTB,),
        in_specs=[
            pl.BlockSpec((TBLK, D), lambda i: (i, 0)),
            pl.BlockSpec((E, D), lambda i: (0, 0)),
        ],
        out_specs=[blk, blk, blk, blk, blk, blk, meta, meta, meta],
        out_shape=[flat_i, flat_i, flat_i, flat_i, flat_f, flat_f,
                   meta_i, meta_i, meta_i],
        scratch_shapes=[
            pltpu.VMEM((128, 1), jnp.float32),
            pltpu.VMEM((128, 1), jnp.float32),
        ],
    )(x2d, router_w)


def _dispatch_body(sel0_h, sel1_h, rank0_h, rank1_h, c0_h, offs_h, x_h,
                   xs_h, dst_h, selc_v, rankc_v, c0_v, offs_v,
                   dst0_v, dst1_v, tok0_v, tok1_v, rows0_v, rows1_v,
                   semg0, semg1, sems0, sems1):
    wid = lax.axis_index("s") * 2 + lax.axis_index("c")
    kflag = wid // 16
    tb = (wid % 16) * 128
    pb = wid * 128

    @pl.when(kflag == 0)
    def _():
        pltpu.sync_copy(sel0_h.at[pl.ds(tb, 128)], selc_v)
        pltpu.sync_copy(rank0_h.at[pl.ds(tb, 128)], rankc_v)

    @pl.when(kflag == 1)
    def _():
        pltpu.sync_copy(sel1_h.at[pl.ds(tb, 128)], selc_v)
        pltpu.sync_copy(rank1_h.at[pl.ds(tb, 128)], rankc_v)

    pltpu.sync_copy(c0_h.at[pl.ds(0, 16)], c0_v)
    pltpu.sync_copy(offs_h.at[pl.ds(0, 16)], offs_v)
    kvec = jnp.full((16,), kflag, dtype=jnp.int32)
    for j in range(8):
        dvh, tvh = (dst0_v, tok0_v) if j < 4 else (dst1_v, tok1_v)
        jh = j % 4
        s16 = selc_v[pl.ds(j * 16, 16)]
        r16 = rankc_v[pl.ds(j * 16, 16)]
        o16 = plsc.load_gather(offs_v, [s16])
        c016 = plsc.load_gather(c0_v, [s16])
        dvh[pl.ds(jh * 16, 16)] = o16 + c016 * kvec + r16
        tvh[pl.ds(jh * 16, 16)] = tb + j * 16 + lax.iota(jnp.int32, 16)
    g0 = pltpu.async_copy(x_h.at[tok0_v], rows0_v, semg0)
    g1 = pltpu.async_copy(x_h.at[tok1_v], rows1_v, semg1)
    g0.wait()
    s0 = pltpu.async_copy(rows0_v, xs_h.at[dst0_v], sems0)
    g1.wait()
    s1 = pltpu.async_copy(rows1_v, xs_h.at[dst1_v], sems1)
    pltpu.sync_copy(dst0_v, dst_h.at[pl.ds(pb, 64)])
    pltpu.sync_copy(dst1_v, dst_h.at[pl.ds(pb + 64, 64)])
    s0.wait()
    s1.wait()


def _dispatch(sel0, sel1, rank0, rank1, c0v, offsv, x2d):
    f = pl.kernel(
        _dispatch_body,
        out_type=[
            jax.ShapeDtypeStruct((P, D), jnp.float32),
            jax.ShapeDtypeStruct((NPAIR,), jnp.int32),
        ],
        mesh=plsc.VectorSubcoreMesh(core_axis_name="c", subcore_axis_name="s"),
        compiler_params=pltpu.CompilerParams(needs_layout_passes=False),
        scratch_types=[
            pltpu.VMEM((128,), jnp.int32),
            pltpu.VMEM((128,), jnp.int32),
            pltpu.VMEM((16,), jnp.int32),
            pltpu.VMEM((16,), jnp.int32),
            pltpu.VMEM((64,), jnp.int32),
            pltpu.VMEM((64,), jnp.int32),
            pltpu.VMEM((64,), jnp.int32),
            pltpu.VMEM((64,), jnp.int32),
            pltpu.VMEM((64, D), jnp.float32),
            pltpu.VMEM((64, D), jnp.float32),
            pltpu.SemaphoreType.DMA,
            pltpu.SemaphoreType.DMA,
            pltpu.SemaphoreType.DMA,
            pltpu.SemaphoreType.DMA,
        ],
    )
    return f(sel0, sel1, rank0, rank1, c0v, offsv, x2d)


def _ffn_body(nb_ref, xs_hbm, w1_hbm, w2_hbm, y_hbm, w1b, w2b, xsb, yb,
              w1sa, w1sb, w2sa, w2sb, xss, yss):
    H = DFFN // 2

    class _Pair:
        def __init__(self, a, b):
            self.a, self.b = a, b

        def start(self):
            self.a.start()
            self.b.start()

        def wait(self):
            self.a.wait()
            self.b.wait()

    def w1cp(e, slot):
        return _Pair(
            pltpu.make_async_copy(w1_hbm.at[:, pl.ds(e * DFFN, H)],
                                  w1b.at[slot, :, pl.ds(0, H)],
                                  w1sa.at[slot]),
            pltpu.make_async_copy(w1_hbm.at[:, pl.ds(e * DFFN + H, H)],
                                  w1b.at[slot, :, pl.ds(H, H)],
                                  w1sb.at[slot]))

    def w2cp(e, slot):
        return _Pair(
            pltpu.make_async_copy(w2_hbm.at[pl.ds(e * DFFN, H), :],
                                  w2b.at[slot, pl.ds(0, H)],
                                  w2sa.at[slot]),
            pltpu.make_async_copy(w2_hbm.at[pl.ds(e * DFFN + H, H), :],
                                  w2b.at[slot, pl.ds(H, H)],
                                  w2sb.at[slot]))

    def xscp(g, slot):
        return pltpu.make_async_copy(
            xs_hbm.at[pl.ds(g * BS, BS)], xsb.at[slot], xss.at[slot])

    def ycp(g, slot):
        return pltpu.make_async_copy(
            yb.at[slot], y_hbm.at[pl.ds(g * BS, BS)], yss.at[slot])

    nbtot = nb_ref[0]
    for e in range(1, E):
        nbtot = nbtot + nb_ref[e]

    for e in range(4):
        w1cp(e, e).start()
        w2cp(e, e).start()
    xscp(0, 0).start()

    g = 0
    for e in range(E):
        slot = e % 4
        w1cp(e, slot).wait()
        w2cp(e, slot).wait()

        def blk(j, g, slot=slot):
            xslot = lax.rem(g, 2)
            xscp(g, xslot).wait()

            @pl.when(g + 1 < nbtot)
            def _():
                xscp(g + 1, lax.rem(g + 1, 2)).start()

            @pl.when(g >= 2)
            def _():
                ycp(g - 2, xslot).wait()

            h = jnp.dot(xsb[xslot].astype(jnp.bfloat16),
                        w1b[slot].astype(jnp.bfloat16),
                        preferred_element_type=jnp.float32)
            h = jax.nn.gelu(h)
            yb[xslot] = jnp.dot(h.astype(jnp.bfloat16),
                                w2b[slot].astype(jnp.bfloat16),
                                preferred_element_type=jnp.float32)
            ycp(g, xslot).start()
            return g + 1

        g = lax.fori_loop(0, nb_ref[e], blk, g)
        if e + 4 < E:
            w1cp(e + 4, slot).start()
            w2cp(e + 4, slot).start()

    ycp(g - 1, lax.rem(g - 1, 2)).wait()
    ycp(g - 2, lax.rem(g - 2, 2)).wait()


def _ffn(nblk, xs, w1, w2):
    return pl.pallas_call(
        _ffn_body,
        in_specs=[
            pl.BlockSpec(memory_space=pltpu.SMEM),
            pl.BlockSpec(memory_space=pl.ANY),
            pl.BlockSpec(memory_space=pl.ANY),
            pl.BlockSpec(memory_space=pl.ANY),
        ],
        out_specs=pl.BlockSpec(memory_space=pl.ANY),
        out_shape=jax.ShapeDtypeStruct((P, D), jnp.float32),
        scratch_shapes=[
            pltpu.VMEM((4, D, DFFN), jnp.float32),
            pltpu.VMEM((4, DFFN, D), jnp.float32),
            pltpu.VMEM((2, BS, D), jnp.float32),
            pltpu.VMEM((2, BS, D), jnp.float32),
            pltpu.SemaphoreType.DMA((4,)),
            pltpu.SemaphoreType.DMA((4,)),
            pltpu.SemaphoreType.DMA((4,)),
            pltpu.SemaphoreType.DMA((4,)),
            pltpu.SemaphoreType.DMA((2,)),
            pltpu.SemaphoreType.DMA((2,)),
        ],
    )(nblk, xs, w1, w2)


def _combine_body(y_h, dst_h, rw0_h, rw1_h, o_h, i0a_v, i1a_v, i0b_v, i1b_v,
                  w0_v, w1_v, r0a_v, r1a_v, r0b_v, r1b_v,
                  sga0, sga1, sgb0, sgb1, soa, sob):
    wid = lax.axis_index("s") * 2 + lax.axis_index("c")
    tb = wid * 64
    pltpu.sync_copy(dst_h.at[pl.ds(tb, 32)], i0a_v)
    pltpu.sync_copy(dst_h.at[pl.ds(T + tb, 32)], i1a_v)
    pltpu.sync_copy(dst_h.at[pl.ds(tb + 32, 32)], i0b_v)
    pltpu.sync_copy(dst_h.at[pl.ds(T + tb + 32, 32)], i1b_v)
    pltpu.sync_copy(rw0_h.at[pl.ds(tb, 64)], w0_v)
    pltpu.sync_copy(rw1_h.at[pl.ds(tb, 64)], w1_v)
    ga0 = pltpu.async_copy(y_h.at[i0a_v], r0a_v, sga0)
    ga1 = pltpu.async_copy(y_h.at[i1a_v], r1a_v, sga1)
    gb0 = pltpu.async_copy(y_h.at[i0b_v], r0b_v, sgb0)
    gb1 = pltpu.async_copy(y_h.at[i1b_v], r1b_v, sgb1)

    def combine(r0_v, r1_v, woff):
        def body(j, carry):
            jw = jnp.full((16,), j + woff, dtype=jnp.int32)
            w0 = plsc.load_gather(w0_v, [jw])
            w1s = plsc.load_gather(w1_v, [jw])
            for c in range(D // 16):
                sl = pl.ds(c * 16, 16)
                r0_v[j, sl] = r0_v[j, sl] * w0 + r1_v[j, sl] * w1s
            return carry
        lax.fori_loop(0, 32, body, 0)

    ga0.wait()
    ga1.wait()
    combine(r0a_v, r1a_v, 0)
    oa = pltpu.async_copy(r0a_v, o_h.at[pl.ds(tb, 32)], soa)
    gb0.wait()
    gb1.wait()
    combine(r0b_v, r1b_v, 32)
    ob = pltpu.async_copy(r0b_v, o_h.at[pl.ds(tb + 32, 32)], sob)
    oa.wait()
    ob.wait()


def _combine(y, dst, rw0f, rw1f):
    f = pl.kernel(
        _combine_body,
        out_type=jax.ShapeDtypeStruct((T, D), jnp.float32),
        mesh=plsc.VectorSubcoreMesh(core_axis_name="c", subcore_axis_name="s"),
        compiler_params=pltpu.CompilerParams(needs_layout_passes=False),
        scratch_types=[
            pltpu.VMEM((32,), jnp.int32),
            pltpu.VMEM((32,), jnp.int32),
            pltpu.VMEM((32,), jnp.int32),
            pltpu.VMEM((32,), jnp.int32),
            pltpu.VMEM((64,), jnp.float32),
            pltpu.VMEM((64,), jnp.float32),
            pltpu.VMEM((32, D), jnp.float32),
            pltpu.VMEM((32, D), jnp.float32),
            pltpu.VMEM((32, D), jnp.float32),
            pltpu.VMEM((32, D), jnp.float32),
            pltpu.SemaphoreType.DMA,
            pltpu.SemaphoreType.DMA,
            pltpu.SemaphoreType.DMA,
            pltpu.SemaphoreType.DMA,
            pltpu.SemaphoreType.DMA,
            pltpu.SemaphoreType.DMA,
        ],
    )
    return f(y, dst, rw0f, rw1f)


def kernel(x, router_w, w1, w2):
    b, s, d = x.shape
    x2d = x.reshape(T, D)
    (sel0, sel1, rank0, rank1, rw0f, rw1f,
     c0v, offsv, nblkv) = _router(x2d, router_w)
    xs, dst = _dispatch(sel0, sel1, rank0, rank1, c0v, offsv, x2d)
    y = _ffn(nblkv, xs, w1, w2)
    out = _combine(y, dst, rw0f, rw1f)
    return out.reshape(b, s, d)


# speedup vs baseline: 1.0652x; 1.0255x over previous
import jax
import jax.numpy as jnp
from jax import lax
from jax.experimental import pallas as pl
from jax.experimental.pallas import tpu as pltpu
from jax.experimental.pallas import tpu_sc as plsc

E = 8
K = 2
D = 768
BS = 256
DFFN = 1536
T = 2048
NPAIR = T * K
NB = 23
P = NB * BS
TBLK = 512
NTB = T // TBLK
NW = 32


def _router_body(x_ref, rwin_ref, sel0_ref, sel1_ref, rank0_ref, rank1_ref,
                 rw0_ref, rw1_ref, c0_ref, offs_ref, nblk_ref,
                 carry0, carry1):
    i = pl.program_id(0)

    @pl.when(i == 0)
    def _():
        carry0[...] = jnp.zeros_like(carry0)
        carry1[...] = jnp.zeros_like(carry1)

    xb = x_ref[...]
    logits = lax.dot_general(rwin_ref[...], xb, (((1,), (1,)), ((), ())),
                             preferred_element_type=jnp.float32)
    m = jnp.max(logits, axis=0, keepdims=True)
    ex = jnp.exp(logits - m)
    p = ex / jnp.sum(ex, axis=0, keepdims=True)
    sub8 = lax.broadcasted_iota(jnp.int32, (E, TBLK), 0)
    m1 = jnp.max(p, axis=0, keepdims=True)
    i1 = jnp.min(jnp.where(p >= m1, sub8, E), axis=0, keepdims=True)
    p2 = jnp.where(sub8 == i1, -1.0, p)
    m2 = jnp.max(p2, axis=0, keepdims=True)
    i2 = jnp.min(jnp.where(p2 >= m2, sub8, E), axis=0, keepdims=True)
    ssum = m1 + m2
    sel0_ref[...] = i1.reshape(TBLK)
    sel1_ref[...] = i2.reshape(TBLK)
    rw0_ref[...] = (m1 / ssum).reshape(TBLK)
    rw1_ref[...] = (m2 / ssum).reshape(TBLK)

    rowE = lax.broadcasted_iota(jnp.int32, (128, TBLK), 0)
    rowT = lax.broadcasted_iota(jnp.int32, (TBLK, TBLK), 0)
    colT = lax.broadcasted_iota(jnp.int32, (TBLK, TBLK), 1)
    triu_incl = (rowT <= colT).astype(jnp.float32)
    for sel, carry, rref in ((i1, carry0, rank0_ref), (i2, carry1, rank1_ref)):
        oh = (rowE == sel).astype(jnp.float32)
        cum = jnp.dot(oh, triu_incl, preferred_element_type=jnp.float32)
        cb = carry[...]
        rank = jnp.sum(oh * (cum + cb - 1.0), axis=0, keepdims=True)
        rref[...] = rank.astype(jnp.int32).reshape(TBLK)
        carry[...] = cb + jnp.sum(oh, axis=1, keepdims=True)

    @pl.when(i == NTB - 1)
    def _():
        row128 = lax.broadcasted_iota(jnp.int32, (128, 128), 0)
        col128 = lax.broadcasted_iota(jnp.int32, (128, 128), 1)
        c0c = carry0[...]
        tot = (c0c + carry1[...]).astype(jnp.int32)
        padded = ((tot + BS - 1) >> 8) << 8
        nblk = (padded >> 8).astype(jnp.float32)
        low_strict = (row128 > col128).astype(jnp.float32)
        offs_col = jnp.dot(low_strict, padded.astype(jnp.float32),
                           preferred_element_type=jnp.float32)
        eye = (row128 == col128).astype(jnp.float32)
        c0_ref[...] = jnp.sum(eye * c0c, axis=0).astype(jnp.int32)
        offs_ref[...] = jnp.sum(eye * offs_col, axis=0).astype(jnp.int32)
        nblk_ref[...] = jnp.sum(eye * nblk, axis=0).astype(jnp.int32)


def _router(x2d, router_w):
    flat_i = jax.ShapeDtypeStruct((T,), jnp.int32)
    flat_f = jax.ShapeDtypeStruct((T,), jnp.float32)
    meta_i = jax.ShapeDtypeStruct((128,), jnp.int32)
    blk = pl.BlockSpec((TBLK,), lambda i: (i,))
    meta = pl.BlockSpec((128,), lambda i: (0,))
    return pl.pallas_call(
        _router_body,
        grid=(NTB,),
        in_specs=[
            pl.BlockSpec((TBLK, D), lambda i: (i, 0)),
            pl.BlockSpec((E, D), lambda i: (0, 0)),
        ],
        out_specs=[blk, blk, blk, blk, blk, blk, meta, meta, meta],
        out_shape=[flat_i, flat_i, flat_i, flat_i, flat_f, flat_f,
                   meta_i, meta_i, meta_i],
        scratch_shapes=[
            pltpu.VMEM((128, 1), jnp.float32),
            pltpu.VMEM((128, 1), jnp.float32),
        ],
    )(x2d, router_w)


def _dispatch_body(sel0_h, sel1_h, rank0_h, rank1_h, c0_h, offs_h, x_h,
                   xs_h, dst_h, selc_v, rankc_v, c0_v, offs_v,
                   dst0_v, dst1_v, tok0_v, tok1_v, rows0_v, rows1_v,
                   semg0, semg1, sems0, sems1):
    wid = lax.axis_index("s") * 2 + lax.axis_index("c")
    kflag = wid // 16
    tb = (wid % 16) * 128
    pb = wid * 128

    @pl.when(kflag == 0)
    def _():
        pltpu.sync_copy(sel0_h.at[pl.ds(tb, 128)], selc_v)
        pltpu.sync_copy(rank0_h.at[pl.ds(tb, 128)], rankc_v)

    @pl.when(kflag == 1)
    def _():
        pltpu.sync_copy(sel1_h.at[pl.ds(tb, 128)], selc_v)
        pltpu.sync_copy(rank1_h.at[pl.ds(tb, 128)], rankc_v)

    pltpu.sync_copy(c0_h.at[pl.ds(0, 16)], c0_v)
    pltpu.sync_copy(offs_h.at[pl.ds(0, 16)], offs_v)
    kvec = jnp.full((16,), kflag, dtype=jnp.int32)
    for j in range(8):
        dvh, tvh = (dst0_v, tok0_v) if j < 4 else (dst1_v, tok1_v)
        jh = j % 4
        s16 = selc_v[pl.ds(j * 16, 16)]
        r16 = rankc_v[pl.ds(j * 16, 16)]
        o16 = plsc.load_gather(offs_v, [s16])
        c016 = plsc.load_gather(c0_v, [s16])
        dvh[pl.ds(jh * 16, 16)] = o16 + c016 * kvec + r16
        tvh[pl.ds(jh * 16, 16)] = tb + j * 16 + lax.iota(jnp.int32, 16)
    g0 = pltpu.async_copy(x_h.at[tok0_v], rows0_v, semg0)
    g1 = pltpu.async_copy(x_h.at[tok1_v], rows1_v, semg1)
    g0.wait()
    s0 = pltpu.async_copy(rows0_v, xs_h.at[dst0_v], sems0)
    g1.wait()
    s1 = pltpu.async_copy(rows1_v, xs_h.at[dst1_v], sems1)
    pltpu.sync_copy(dst0_v, dst_h.at[pl.ds(pb, 64)])
    pltpu.sync_copy(dst1_v, dst_h.at[pl.ds(pb + 64, 64)])
    s0.wait()
    s1.wait()


def _dispatch(sel0, sel1, rank0, rank1, c0v, offsv, x2d):
    f = pl.kernel(
        _dispatch_body,
        out_type=[
            jax.ShapeDtypeStruct((P, D), jnp.float32),
            jax.ShapeDtypeStruct((NPAIR,), jnp.int32),
        ],
        mesh=plsc.VectorSubcoreMesh(core_axis_name="c", subcore_axis_name="s"),
        compiler_params=pltpu.CompilerParams(needs_layout_passes=False),
        scratch_types=[
            pltpu.VMEM((128,), jnp.int32),
            pltpu.VMEM((128,), jnp.int32),
            pltpu.VMEM((16,), jnp.int32),
            pltpu.VMEM((16,), jnp.int32),
            pltpu.VMEM((64,), jnp.int32),
            pltpu.VMEM((64,), jnp.int32),
            pltpu.VMEM((64,), jnp.int32),
            pltpu.VMEM((64,), jnp.int32),
            pltpu.VMEM((64, D), jnp.float32),
            pltpu.VMEM((64, D), jnp.float32),
            pltpu.SemaphoreType.DMA,
            pltpu.SemaphoreType.DMA,
            pltpu.SemaphoreType.DMA,
            pltpu.SemaphoreType.DMA,
        ],
    )
    return f(sel0, sel1, rank0, rank1, c0v, offsv, x2d)


def _ffn_body(nb_ref, xs_hbm, w1_hbm, w2_hbm, y_hbm, w1b, w2b, xsb, yb,
              w1sa, w1sb, w2sa, w2sb, xss, yss):
    H = DFFN // 2

    class _Pair:
        def __init__(self, a, b):
            self.a, self.b = a, b

        def start(self):
            self.a.start()
            self.b.start()

        def wait(self):
            self.a.wait()
            self.b.wait()

    def w1cp(e, slot):
        return _Pair(
            pltpu.make_async_copy(w1_hbm.at[:, pl.ds(e * DFFN, H)],
                                  w1b.at[slot, :, pl.ds(0, H)],
                                  w1sa.at[slot]),
            pltpu.make_async_copy(w1_hbm.at[:, pl.ds(e * DFFN + H, H)],
                                  w1b.at[slot, :, pl.ds(H, H)],
                                  w1sb.at[slot]))

    def w2cp(e, slot):
        return _Pair(
            pltpu.make_async_copy(w2_hbm.at[pl.ds(e * DFFN, H), :],
                                  w2b.at[slot, pl.ds(0, H)],
                                  w2sa.at[slot]),
            pltpu.make_async_copy(w2_hbm.at[pl.ds(e * DFFN + H, H), :],
                                  w2b.at[slot, pl.ds(H, H)],
                                  w2sb.at[slot]))

    def xscp(g, slot):
        return pltpu.make_async_copy(
            xs_hbm.at[pl.ds(g * BS, BS)], xsb.at[slot], xss.at[slot])

    def ycp(g, slot):
        return pltpu.make_async_copy(
            yb.at[slot], y_hbm.at[pl.ds(g * BS, BS)], yss.at[slot])

    nbtot = nb_ref[0]
    for e in range(1, E):
        nbtot = nbtot + nb_ref[e]

    for e in range(4):
        w1cp(e, e).start()
        w2cp(e, e).start()
    xscp(0, 0).start()

    g = 0
    for e in range(E):
        slot = e % 4
        w1cp(e, slot).wait()
        w2cp(e, slot).wait()

        def blk(j, g, slot=slot):
            xslot = lax.rem(g, 2)
            xscp(g, xslot).wait()

            @pl.when(g + 1 < nbtot)
            def _():
                xscp(g + 1, lax.rem(g + 1, 2)).start()

            @pl.when(g >= 2)
            def _():
                ycp(g - 2, xslot).wait()

            h = jnp.dot(xsb[xslot].astype(jnp.bfloat16),
                        w1b[slot].astype(jnp.bfloat16),
                        preferred_element_type=jnp.float32)
            h = jax.nn.gelu(h)
            yb[xslot] = jnp.dot(h.astype(jnp.bfloat16),
                                w2b[slot].astype(jnp.bfloat16),
                                preferred_element_type=jnp.float32)
            ycp(g, xslot).start()
            return g + 1

        g = lax.fori_loop(0, nb_ref[e], blk, g)
        if e + 4 < E:
            w1cp(e + 4, slot).start()
            w2cp(e + 4, slot).start()

    ycp(g - 1, lax.rem(g - 1, 2)).wait()
    ycp(g - 2, lax.rem(g - 2, 2)).wait()


def _ffn(nblk, xs, w1, w2):
    return pl.pallas_call(
        _ffn_body,
        in_specs=[
            pl.BlockSpec(memory_space=pltpu.SMEM),
            pl.BlockSpec(memory_space=pl.ANY),
            pl.BlockSpec(memory_space=pl.ANY),
            pl.BlockSpec(memory_space=pl.ANY),
        ],
        out_specs=pl.BlockSpec(memory_space=pl.ANY),
        out_shape=jax.ShapeDtypeStruct((P, D), jnp.float32),
        scratch_shapes=[
            pltpu.VMEM((4, D, DFFN), jnp.float32),
            pltpu.VMEM((4, DFFN, D), jnp.float32),
            pltpu.VMEM((2, BS, D), jnp.float32),
            pltpu.VMEM((2, BS, D), jnp.float32),
            pltpu.SemaphoreType.DMA((4,)),
            pltpu.SemaphoreType.DMA((4,)),
            pltpu.SemaphoreType.DMA((4,)),
            pltpu.SemaphoreType.DMA((4,)),
            pltpu.SemaphoreType.DMA((2,)),
            pltpu.SemaphoreType.DMA((2,)),
        ],
    )(nblk, xs, w1, w2)


def _combine_body(y_h, dst_h, rw0_h, rw1_h, o_h, i0a_v, i1a_v, i0b_v, i1b_v,
                  w0_v, w1_v, r0a_v, r1a_v, r0b_v, r1b_v,
                  sga0, sga1, sgb0, sgb1, soa, sob):
    wid = lax.axis_index("s") * 2 + lax.axis_index("c")
    tb = wid * 64
    pltpu.sync_copy(dst_h.at[pl.ds(tb, 32)], i0a_v)
    pltpu.sync_copy(dst_h.at[pl.ds(T + tb, 32)], i1a_v)
    pltpu.sync_copy(dst_h.at[pl.ds(tb + 32, 32)], i0b_v)
    pltpu.sync_copy(dst_h.at[pl.ds(T + tb + 32, 32)], i1b_v)
    pltpu.sync_copy(rw0_h.at[pl.ds(tb, 64)], w0_v)
    pltpu.sync_copy(rw1_h.at[pl.ds(tb, 64)], w1_v)
    ga0 = pltpu.async_copy(y_h.at[i0a_v], r0a_v, sga0)
    ga1 = pltpu.async_copy(y_h.at[i1a_v], r1a_v, sga1)
    gb0 = pltpu.async_copy(y_h.at[i0b_v], r0b_v, sgb0)
    gb1 = pltpu.async_copy(y_h.at[i1b_v], r1b_v, sgb1)

    def combine(r0_v, r1_v, woff):
        def body(j, carry):
            jw = jnp.full((16,), j + woff, dtype=jnp.int32)
            w0 = plsc.load_gather(w0_v, [jw])
            w1s = plsc.load_gather(w1_v, [jw])
            for c in range(D // 16):
                sl = pl.ds(c * 16, 16)
                r0_v[j, sl] = r0_v[j, sl] * w0 + r1_v[j, sl] * w1s
            return carry
        lax.fori_loop(0, 32, body, 0)

    ga0.wait()
    ga1.wait()
    combine(r0a_v, r1a_v, 0)
    oa = pltpu.async_copy(r0a_v, o_h.at[pl.ds(tb, 32)], soa)
    gb0.wait()
    gb1.wait()
    combine(r0b_v, r1b_v, 32)
    ob = pltpu.async_copy(r0b_v, o_h.at[pl.ds(tb + 32, 32)], sob)
    oa.wait()
    ob.wait()


def _combine(y, dst, rw0f, rw1f):
    f = pl.kernel(
        _combine_body,
        out_type=jax.ShapeDtypeStruct((T, D), jnp.float32),
        mesh=plsc.VectorSubcoreMesh(core_axis_name="c", subcore_axis_name="s"),
        compiler_params=pltpu.CompilerParams(needs_layout_passes=False),
        scratch_types=[
            pltpu.VMEM((32,), jnp.int32),
            pltpu.VMEM((32,), jnp.int32),
            pltpu.VMEM((32,), jnp.int32),
            pltpu.VMEM((32,), jnp.int32),
            pltpu.VMEM((64,), jnp.float32),
            pltpu.VMEM((64,), jnp.float32),
            pltpu.VMEM((32, D), jnp.float32),
            pltpu.VMEM((32, D), jnp.float32),
            pltpu.VMEM((32, D), jnp.float32),
            pltpu.VMEM((32, D), jnp.float32),
            pltpu.SemaphoreType.DMA,
            pltpu.SemaphoreType.DMA,
            pltpu.SemaphoreType.DMA,
            pltpu.SemaphoreType.DMA,
            pltpu.SemaphoreType.DMA,
            pltpu.SemaphoreType.DMA,
        ],
    )
    return f(y, dst, rw0f, rw1f)


def kernel(x, router_w, w1, w2):
    b, s, d = x.shape
    x2d = x.reshape(T, D)
    (sel0, sel1, rank0, rank1, rw0f, rw1f,
     c0v, offsv, nblkv) = _router(x2d, router_w)
    xs, dst = _dispatch(sel0, sel1, rank0, rank1, c0v, offsv, x2d)
    y = _ffn(nblkv, xs, w1, w2)
    out = _combine(y, dst, rw0f, rw1f)
    return out.reshape(b, s, d)


# speedup vs baseline: 1.0703x; 1.0048x over previous
import jax
import jax.numpy as jnp
from jax import lax
from jax.experimental import pallas as pl
from jax.experimental.pallas import tpu as pltpu
from jax.experimental.pallas import tpu_sc as plsc

E = 8
K = 2
D = 768
BS = 256
DFFN = 1536
T = 2048
NPAIR = T * K
NB = 23
P = NB * BS
TBLK = 1024
NTB = T // TBLK
NW = 32


def _router_body(x_ref, rwin_ref, sel0_ref, sel1_ref, rank0_ref, rank1_ref,
                 rw0_ref, rw1_ref, c0_ref, offs_ref, nblk_ref,
                 carry0, carry1):
    i = pl.program_id(0)

    @pl.when(i == 0)
    def _():
        carry0[...] = jnp.zeros_like(carry0)
        carry1[...] = jnp.zeros_like(carry1)

    xb = x_ref[...]
    logits = lax.dot_general(rwin_ref[...], xb, (((1,), (1,)), ((), ())),
                             preferred_element_type=jnp.float32)
    m = jnp.max(logits, axis=0, keepdims=True)
    ex = jnp.exp(logits - m)
    p = ex / jnp.sum(ex, axis=0, keepdims=True)
    sub8 = lax.broadcasted_iota(jnp.int32, (E, TBLK), 0)
    m1 = jnp.max(p, axis=0, keepdims=True)
    i1 = jnp.min(jnp.where(p >= m1, sub8, E), axis=0, keepdims=True)
    p2 = jnp.where(sub8 == i1, -1.0, p)
    m2 = jnp.max(p2, axis=0, keepdims=True)
    i2 = jnp.min(jnp.where(p2 >= m2, sub8, E), axis=0, keepdims=True)
    ssum = m1 + m2
    sel0_ref[...] = i1.reshape(TBLK)
    sel1_ref[...] = i2.reshape(TBLK)
    rw0_ref[...] = (m1 / ssum).reshape(TBLK)
    rw1_ref[...] = (m2 / ssum).reshape(TBLK)

    rowE = lax.broadcasted_iota(jnp.int32, (128, TBLK), 0)
    rowT = lax.broadcasted_iota(jnp.int32, (TBLK, TBLK), 0)
    colT = lax.broadcasted_iota(jnp.int32, (TBLK, TBLK), 1)
    triu_incl = (rowT <= colT).astype(jnp.float32)
    for sel, carry, rref in ((i1, carry0, rank0_ref), (i2, carry1, rank1_ref)):
        oh = (rowE == sel).astype(jnp.float32)
        cum = jnp.dot(oh, triu_incl, preferred_element_type=jnp.float32)
        cb = carry[...]
        rank = jnp.sum(oh * (cum + cb - 1.0), axis=0, keepdims=True)
        rref[...] = rank.astype(jnp.int32).reshape(TBLK)
        carry[...] = cb + jnp.sum(oh, axis=1, keepdims=True)

    @pl.when(i == NTB - 1)
    def _():
        row128 = lax.broadcasted_iota(jnp.int32, (128, 128), 0)
        col128 = lax.broadcasted_iota(jnp.int32, (128, 128), 1)
        c0c = carry0[...]
        tot = (c0c + carry1[...]).astype(jnp.int32)
        padded = ((tot + BS - 1) >> 8) << 8
        nblk = (padded >> 8).astype(jnp.float32)
        low_strict = (row128 > col128).astype(jnp.float32)
        offs_col = jnp.dot(low_strict, padded.astype(jnp.float32),
                           preferred_element_type=jnp.float32)
        eye = (row128 == col128).astype(jnp.float32)
        c0_ref[...] = jnp.sum(eye * c0c, axis=0).astype(jnp.int32)
        offs_ref[...] = jnp.sum(eye * offs_col, axis=0).astype(jnp.int32)
        nblk_ref[...] = jnp.sum(eye * nblk, axis=0).astype(jnp.int32)


def _router(x2d, router_w):
    flat_i = jax.ShapeDtypeStruct((T,), jnp.int32)
    flat_f = jax.ShapeDtypeStruct((T,), jnp.float32)
    meta_i = jax.ShapeDtypeStruct((128,), jnp.int32)
    blk = pl.BlockSpec((TBLK,), lambda i: (i,))
    meta = pl.BlockSpec((128,), lambda i: (0,))
    return pl.pallas_call(
        _router_body,
        grid=(NTB,),
        in_specs=[
            pl.BlockSpec((TBLK, D), lambda i: (i, 0)),
            pl.BlockSpec((E, D), lambda i: (0, 0)),
        ],
        out_specs=[blk, blk, blk, blk, blk, blk, meta, meta, meta],
        out_shape=[flat_i, flat_i, flat_i, flat_i, flat_f, flat_f,
                   meta_i, meta_i, meta_i],
        scratch_shapes=[
            pltpu.VMEM((128, 1), jnp.float32),
            pltpu.VMEM((128, 1), jnp.float32),
        ],
    )(x2d, router_w)


def _dispatch_body(sel0_h, sel1_h, rank0_h, rank1_h, c0_h, offs_h, x_h,
                   xs_h, dst_h, selc_v, rankc_v, c0_v, offs_v,
                   dst0_v, dst1_v, tok0_v, tok1_v, rows0_v, rows1_v,
                   semg0, semg1, sems0, sems1):
    wid = lax.axis_index("s") * 2 + lax.axis_index("c")
    kflag = wid // 16
    tb = (wid % 16) * 128
    pb = wid * 128

    @pl.when(kflag == 0)
    def _():
        pltpu.sync_copy(sel0_h.at[pl.ds(tb, 128)], selc_v)
        pltpu.sync_copy(rank0_h.at[pl.ds(tb, 128)], rankc_v)

    @pl.when(kflag == 1)
    def _():
        pltpu.sync_copy(sel1_h.at[pl.ds(tb, 128)], selc_v)
        pltpu.sync_copy(rank1_h.at[pl.ds(tb, 128)], rankc_v)

    pltpu.sync_copy(c0_h.at[pl.ds(0, 16)], c0_v)
    pltpu.sync_copy(offs_h.at[pl.ds(0, 16)], offs_v)
    kvec = jnp.full((16,), kflag, dtype=jnp.int32)
    for j in range(8):
        dvh, tvh = (dst0_v, tok0_v) if j < 4 else (dst1_v, tok1_v)
        jh = j % 4
        s16 = selc_v[pl.ds(j * 16, 16)]
        r16 = rankc_v[pl.ds(j * 16, 16)]
        o16 = plsc.load_gather(offs_v, [s16])
        c016 = plsc.load_gather(c0_v, [s16])
        dvh[pl.ds(jh * 16, 16)] = o16 + c016 * kvec + r16
        tvh[pl.ds(jh * 16, 16)] = tb + j * 16 + lax.iota(jnp.int32, 16)
    g0 = pltpu.async_copy(x_h.at[tok0_v], rows0_v, semg0)
    g1 = pltpu.async_copy(x_h.at[tok1_v], rows1_v, semg1)
    g0.wait()
    s0 = pltpu.async_copy(rows0_v, xs_h.at[dst0_v], sems0)
    g1.wait()
    s1 = pltpu.async_copy(rows1_v, xs_h.at[dst1_v], sems1)
    pltpu.sync_copy(dst0_v, dst_h.at[pl.ds(pb, 64)])
    pltpu.sync_copy(dst1_v, dst_h.at[pl.ds(pb + 64, 64)])
    s0.wait()
    s1.wait()


def _dispatch(sel0, sel1, rank0, rank1, c0v, offsv, x2d):
    f = pl.kernel(
        _dispatch_body,
        out_type=[
            jax.ShapeDtypeStruct((P, D), jnp.float32),
            jax.ShapeDtypeStruct((NPAIR,), jnp.int32),
        ],
        mesh=plsc.VectorSubcoreMesh(core_axis_name="c", subcore_axis_name="s"),
        compiler_params=pltpu.CompilerParams(needs_layout_passes=False),
        scratch_types=[
            pltpu.VMEM((128,), jnp.int32),
            pltpu.VMEM((128,), jnp.int32),
            pltpu.VMEM((16,), jnp.int32),
            pltpu.VMEM((16,), jnp.int32),
            pltpu.VMEM((64,), jnp.int32),
            pltpu.VMEM((64,), jnp.int32),
            pltpu.VMEM((64,), jnp.int32),
            pltpu.VMEM((64,), jnp.int32),
            pltpu.VMEM((64, D), jnp.float32),
            pltpu.VMEM((64, D), jnp.float32),
            pltpu.SemaphoreType.DMA,
            pltpu.SemaphoreType.DMA,
            pltpu.SemaphoreType.DMA,
            pltpu.SemaphoreType.DMA,
        ],
    )
    return f(sel0, sel1, rank0, rank1, c0v, offsv, x2d)


def _ffn_body(nb_ref, xs_hbm, w1_hbm, w2_hbm, y_hbm, w1b, w2b, xsb, yb,
              w1sa, w1sb, w2sa, w2sb, xss, yss):
    H = DFFN // 2

    class _Pair:
        def __init__(self, a, b):
            self.a, self.b = a, b

        def start(self):
            self.a.start()
            self.b.start()

        def wait(self):
            self.a.wait()
            self.b.wait()

    def w1cp(e, slot):
        return _Pair(
            pltpu.make_async_copy(w1_hbm.at[:, pl.ds(e * DFFN, H)],
                                  w1b.at[slot, :, pl.ds(0, H)],
                                  w1sa.at[slot]),
            pltpu.make_async_copy(w1_hbm.at[:, pl.ds(e * DFFN + H, H)],
                                  w1b.at[slot, :, pl.ds(H, H)],
                                  w1sb.at[slot]))

    def w2cp(e, slot):
        return _Pair(
            pltpu.make_async_copy(w2_hbm.at[pl.ds(e * DFFN, H), :],
                                  w2b.at[slot, pl.ds(0, H)],
                                  w2sa.at[slot]),
            pltpu.make_async_copy(w2_hbm.at[pl.ds(e * DFFN + H, H), :],
                                  w2b.at[slot, pl.ds(H, H)],
                                  w2sb.at[slot]))

    def xscp(g, slot):
        return pltpu.make_async_copy(
            xs_hbm.at[pl.ds(g * BS, BS)], xsb.at[slot], xss.at[slot])

    def ycp(g, slot):
        return pltpu.make_async_copy(
            yb.at[slot], y_hbm.at[pl.ds(g * BS, BS)], yss.at[slot])

    nbtot = nb_ref[0]
    for e in range(1, E):
        nbtot = nbtot + nb_ref[e]

    for e in range(4):
        w1cp(e, e).start()
        w2cp(e, e).start()
    xscp(0, 0).start()

    g = 0
    for e in range(E):
        slot = e % 4
        w1cp(e, slot).wait()
        w2cp(e, slot).wait()

        def blk(j, g, slot=slot):
            xslot = lax.rem(g, 2)
            xscp(g, xslot).wait()

            @pl.when(g + 1 < nbtot)
            def _():
                xscp(g + 1, lax.rem(g + 1, 2)).start()

            @pl.when(g >= 2)
            def _():
                ycp(g - 2, xslot).wait()

            h = jnp.dot(xsb[xslot].astype(jnp.bfloat16),
                        w1b[slot].astype(jnp.bfloat16),
                        preferred_element_type=jnp.float32)
            h = jax.nn.gelu(h)
            yb[xslot] = jnp.dot(h.astype(jnp.bfloat16),
                                w2b[slot].astype(jnp.bfloat16),
                                preferred_element_type=jnp.float32)
            ycp(g, xslot).start()
            return g + 1

        g = lax.fori_loop(0, nb_ref[e], blk, g)
        if e + 4 < E:
            w1cp(e + 4, slot).start()
            w2cp(e + 4, slot).start()

    ycp(g - 1, lax.rem(g - 1, 2)).wait()
    ycp(g - 2, lax.rem(g - 2, 2)).wait()


def _ffn(nblk, xs, w1, w2):
    return pl.pallas_call(
        _ffn_body,
        in_specs=[
            pl.BlockSpec(memory_space=pltpu.SMEM),
            pl.BlockSpec(memory_space=pl.ANY),
            pl.BlockSpec(memory_space=pl.ANY),
            pl.BlockSpec(memory_space=pl.ANY),
        ],
        out_specs=pl.BlockSpec(memory_space=pl.ANY),
        out_shape=jax.ShapeDtypeStruct((P, D), jnp.float32),
        scratch_shapes=[
            pltpu.VMEM((4, D, DFFN), jnp.float32),
            pltpu.VMEM((4, DFFN, D), jnp.float32),
            pltpu.VMEM((2, BS, D), jnp.float32),
            pltpu.VMEM((2, BS, D), jnp.float32),
            pltpu.SemaphoreType.DMA((4,)),
            pltpu.SemaphoreType.DMA((4,)),
            pltpu.SemaphoreType.DMA((4,)),
            pltpu.SemaphoreType.DMA((4,)),
            pltpu.SemaphoreType.DMA((2,)),
            pltpu.SemaphoreType.DMA((2,)),
        ],
    )(nblk, xs, w1, w2)


def _combine_body(y_h, dst_h, rw0_h, rw1_h, o_h, i0a_v, i1a_v, i0b_v, i1b_v,
                  w0_v, w1_v, r0a_v, r1a_v, r0b_v, r1b_v,
                  sga0, sga1, sgb0, sgb1, soa, sob):
    wid = lax.axis_index("s") * 2 + lax.axis_index("c")
    tb = wid * 64
    pltpu.sync_copy(dst_h.at[pl.ds(tb, 32)], i0a_v)
    pltpu.sync_copy(dst_h.at[pl.ds(T + tb, 32)], i1a_v)
    pltpu.sync_copy(dst_h.at[pl.ds(tb + 32, 32)], i0b_v)
    pltpu.sync_copy(dst_h.at[pl.ds(T + tb + 32, 32)], i1b_v)
    pltpu.sync_copy(rw0_h.at[pl.ds(tb, 64)], w0_v)
    pltpu.sync_copy(rw1_h.at[pl.ds(tb, 64)], w1_v)
    ga0 = pltpu.async_copy(y_h.at[i0a_v], r0a_v, sga0)
    ga1 = pltpu.async_copy(y_h.at[i1a_v], r1a_v, sga1)
    gb0 = pltpu.async_copy(y_h.at[i0b_v], r0b_v, sgb0)
    gb1 = pltpu.async_copy(y_h.at[i1b_v], r1b_v, sgb1)

    def combine(r0_v, r1_v, woff):
        def body(j, carry):
            jw = jnp.full((16,), j + woff, dtype=jnp.int32)
            w0 = plsc.load_gather(w0_v, [jw])
            w1s = plsc.load_gather(w1_v, [jw])
            for c in range(D // 16):
                sl = pl.ds(c * 16, 16)
                r0_v[j, sl] = r0_v[j, sl] * w0 + r1_v[j, sl] * w1s
            return carry
        lax.fori_loop(0, 32, body, 0)

    ga0.wait()
    ga1.wait()
    combine(r0a_v, r1a_v, 0)
    oa = pltpu.async_copy(r0a_v, o_h.at[pl.ds(tb, 32)], soa)
    gb0.wait()
    gb1.wait()
    combine(r0b_v, r1b_v, 32)
    ob = pltpu.async_copy(r0b_v, o_h.at[pl.ds(tb + 32, 32)], sob)
    oa.wait()
    ob.wait()


def _combine(y, dst, rw0f, rw1f):
    f = pl.kernel(
        _combine_body,
        out_type=jax.ShapeDtypeStruct((T, D), jnp.float32),
        mesh=plsc.VectorSubcoreMesh(core_axis_name="c", subcore_axis_name="s"),
        compiler_params=pltpu.CompilerParams(needs_layout_passes=False),
        scratch_types=[
            pltpu.VMEM((32,), jnp.int32),
            pltpu.VMEM((32,), jnp.int32),
            pltpu.VMEM((32,), jnp.int32),
            pltpu.VMEM((32,), jnp.int32),
            pltpu.VMEM((64,), jnp.float32),
            pltpu.VMEM((64,), jnp.float32),
            pltpu.VMEM((32, D), jnp.float32),
            pltpu.VMEM((32, D), jnp.float32),
            pltpu.VMEM((32, D), jnp.float32),
            pltpu.VMEM((32, D), jnp.float32),
            pltpu.SemaphoreType.DMA,
            pltpu.SemaphoreType.DMA,
            pltpu.SemaphoreType.DMA,
            pltpu.SemaphoreType.DMA,
            pltpu.SemaphoreType.DMA,
            pltpu.SemaphoreType.DMA,
        ],
    )
    return f(y, dst, rw0f, rw1f)


def kernel(x, router_w, w1, w2):
    b, s, d = x.shape
    x2d = x.reshape(T, D)
    (sel0, sel1, rank0, rank1, rw0f, rw1f,
     c0v, offsv, nblkv) = _router(x2d, router_w)
    xs, dst = _dispatch(sel0, sel1, rank0, rank1, c0v, offsv, x2d)
    y = _ffn(nblkv, xs, w1, w2)
    out = _combine(y, dst, rw0f, rw1f)
    return out.reshape(b, s, d)
